# baseline (speedup 1.0000x reference)
import functools

import numpy as np
import jax
import jax.numpy as jnp
from jax import lax
from jax.experimental import pallas as pl
from jax.experimental.pallas import tpu as pltpu
from jax.experimental.pallas import tpu_sc as plsc

_GAMMA = 10.0
_GRID = 64
_THRESH = 1.0
_SIGMA = 1.5
_CONG_W = 0.1
_STEEP = 2.0

_B, _V, _P, _N, _MP = 4, 512, 8192, 2048, 16
_NW = 32
_WPB = _NW // _B
_NPW = _N // _WPB
_NGRP = _NPW // 16
_CSH = float(_GRID // 2)


def _gauss_band_matrix():
    ksize = max(int(4 * _SIGMA) | 1, 3)
    half = ksize // 2
    x = np.arange(ksize, dtype=np.float64) - half
    g1 = np.exp(-(x ** 2) / (2 * _SIGMA ** 2))
    g1 = g1 / g1.sum()
    A = np.zeros((_GRID, _GRID), dtype=np.float32)
    for i in range(_GRID):
        for j in range(max(0, i - half), min(_GRID, i + half + 1)):
            A[i, j] = g1[j - i + half]
    return A


_GAUSS_A = jnp.asarray(_gauss_band_matrix())


_OFF_POS = 0
_OFF_OFF = _OFF_POS + _B * _V * 2
_OFF_OH = _OFF_OFF + _P * 2
_OFF_NTP = _OFF_OH + _B * _V * 4
_OFF_P2M = _OFF_NTP + _N * _MP
_FLEN = _OFF_P2M + _P


def _sc_stats_body(fbuf_hbm, out_hbm,
                   ntp_v, p2m_v, off_v, pos_v, oh_v, c_v, s_v, res_v):
    wid = lax.axis_index("s") * 2 + lax.axis_index("c")
    b = wid // _WPB
    n0 = (wid % _WPB) * _NPW

    pltpu.sync_copy(fbuf_hbm.at[pl.ds(_OFF_NTP + n0 * _MP, _NPW * _MP)],
                    ntp_v)
    pltpu.sync_copy(fbuf_hbm.at[pl.ds(_OFF_P2M, _P)], p2m_v)
    pltpu.sync_copy(fbuf_hbm.at[pl.ds(_OFF_OFF, _P * 2)], off_v)
    pltpu.sync_copy(fbuf_hbm.at[pl.ds(_OFF_POS + b * _V * 2, _V * 2)], pos_v)
    pltpu.sync_copy(fbuf_hbm.at[pl.ds(_OFF_OH + b * _V * 4, _V * 4)], oh_v)

    lanes = lax.broadcasted_iota(jnp.int32, (16,), 0)
    neg = jnp.full((16,), -1e9, jnp.float32)
    pos = jnp.full((16,), 1e9, jnp.float32)

    @plsc.parallel_loop(0, _V // 16, unroll=2)
    def build_cs(j):
        i4 = (j * 16 + lanes) * 4
        o0 = plsc.load_gather(oh_v, [i4])
        o1 = plsc.load_gather(oh_v, [i4 + 1])
        o2 = plsc.load_gather(oh_v, [i4 + 2])
        o3 = plsc.load_gather(oh_v, [i4 + 3])
        sl = pl.ds(j * 16, 16)
        c_v[sl] = o0 - o2
        s_v[sl] = o1 - o3

    @plsc.parallel_loop(0, _NGRP, unroll=4)
    def group(t):
        row = t * (16 * _MP) + lanes * _MP
        nsl = pl.ds(t * 16, 16)
        zero = jnp.zeros((16,), jnp.float32)
        sgx, snx, sgy, sny = zero, zero, zero, zero
        bxmax, bymax = neg, neg
        bxmin, bymin = pos, pos
        for m in range(_MP):
            idx = plsc.bitcast(plsc.load_gather(ntp_v, [row + m]), jnp.int32)
            valid = idx >= 0
            safe = jnp.maximum(idx, 0)
            mac = plsc.bitcast(plsc.load_gather(p2m_v, [safe]), jnp.int32)
            mac2 = mac + mac
            px = plsc.load_gather(pos_v, [mac2])
            py = plsc.load_gather(pos_v, [mac2 + 1])
            c = plsc.load_gather(c_v, [mac])
            s = plsc.load_gather(s_v, [mac])
            safe2 = safe + safe
            ox = plsc.load_gather(off_v, [safe2])
            oy = plsc.load_gather(off_v, [safe2 + 1])
            x = px + c * ox - s * oy
            y = py + s * ox + c * oy
            xm = jnp.where(valid, x, neg)
            xn = jnp.where(valid, x, pos)
            ym = jnp.where(valid, y, neg)
            yn = jnp.where(valid, y, pos)
            sgx = sgx + jnp.exp(_GAMMA * xm)
            snx = snx + jnp.exp(-_GAMMA * xn)
            sgy = sgy + jnp.exp(_GAMMA * ym)
            sny = sny + jnp.exp(-_GAMMA * yn)
            bxmax = jnp.maximum(bxmax, xm)
            bxmin = jnp.minimum(bxmin, xn)
            bymax = jnp.maximum(bymax, ym)
            bymin = jnp.minimum(bymin, yn)
        res_v[0, nsl] = sgx
        res_v[1, nsl] = snx
        res_v[2, nsl] = sgy
        res_v[3, nsl] = sny
        res_v[4, nsl] = bxmax
        res_v[5, nsl] = bxmin
        res_v[6, nsl] = bymax
        res_v[7, nsl] = bymin

    for k in range(8):
        pltpu.sync_copy(res_v.at[k], out_hbm.at[b * 8 + k, pl.ds(n0, _NPW)])


@functools.lru_cache(maxsize=1)
def _sc_stats():
    return pl.kernel(
        _sc_stats_body,
        mesh=plsc.VectorSubcoreMesh(core_axis_name="c", subcore_axis_name="s"),
        compiler_params=pltpu.CompilerParams(needs_layout_passes=False),
        out_type=jax.ShapeDtypeStruct((8 * _B, _N), jnp.float32),
        scratch_types=[
            pltpu.VMEM((_NPW * _MP,), jnp.float32),
            pltpu.VMEM((_P,), jnp.float32),
            pltpu.VMEM((_P * 2,), jnp.float32),
            pltpu.VMEM((_V * 2,), jnp.float32),
            pltpu.VMEM((_V * 4,), jnp.float32),
            pltpu.VMEM((_V,), jnp.float32),
            pltpu.VMEM((_V,), jnp.float32),
            pltpu.VMEM((8, _NPW), jnp.float32),
        ],
    )


def _tc_tail_body(sums_ref, w_ref, A_ref, tot_ref, hpwl_ref, pen_ref):
    A = A_ref[...]
    w = w_ref[...]
    irow = lax.broadcasted_iota(jnp.int32, (_GRID, 1), 0).astype(jnp.float32)
    u = jnp.exp(_STEEP * (_CSH - irow))
    v = jnp.exp(_STEEP * (irow - _CSH))

    for b in range(_B):
        S = sums_ref[b * 8:(b + 1) * 8, :]
        logs = jnp.log(S[0:4, :])
        wl = jnp.sum(logs, axis=0, keepdims=True) * (1.0 / _GAMMA)
        hp = jnp.sum(wl * w)

        scale = 0.5 * (_GRID - 1)
        gxmax = (S[4:5, :] + 1.0) * scale
        gxmin = (S[5:6, :] + 1.0) * scale
        gymax = (S[6:7, :] + 1.0) * scale
        gymin = (S[7:8, :] + 1.0) * scale

        eax = jnp.exp(_STEEP * (gxmin - _CSH))
        ebx = jnp.exp(-_STEEP * (gxmax - _CSH))
        eay = jnp.exp(_STEEP * (gymin - _CSH))
        eby = jnp.exp(-_STEEP * (gymax - _CSH))
        wx = 1.0 / ((1.0 + eax * ebx) + (eax * u + ebx * v))
        wy = 1.0 / ((1.0 + eay * eby) + (eay * u + eby * v))
        area = jnp.clip((gxmax - gxmin + 1.0) * (gymax - gymin + 1.0),
                        1.0, None)
        wxa = wx * (1.0 / area)

        rudy = lax.dot_general(
            wy, wxa, (((1,), (1,)), ((), ())),
            preferred_element_type=jnp.float32,
            precision=lax.Precision.HIGHEST)
        sm = jnp.dot(A, rudy, preferred_element_type=jnp.float32,
                     precision=lax.Precision.HIGHEST)
        sm = jnp.dot(sm, A, preferred_element_type=jnp.float32,
                     precision=lax.Precision.HIGHEST)
        o = jnp.maximum(sm - _THRESH, 0.0)
        pen = jnp.sum(o * o)

        hpwl_ref[b] = hp
        pen_ref[b] = pen
        tot_ref[b] = hp + _CONG_W * pen


def _tc_tail(sums, weights, A):
    return pl.pallas_call(
        _tc_tail_body,
        out_shape=(
            jax.ShapeDtypeStruct((_B,), jnp.float32),
            jax.ShapeDtypeStruct((_B,), jnp.float32),
            jax.ShapeDtypeStruct((_B,), jnp.float32),
        ),
        in_specs=[
            pl.BlockSpec(memory_space=pltpu.VMEM),
            pl.BlockSpec(memory_space=pltpu.VMEM),
            pl.BlockSpec(memory_space=pltpu.VMEM),
        ],
        out_specs=(
            pl.BlockSpec(memory_space=pltpu.SMEM),
            pl.BlockSpec(memory_space=pltpu.SMEM),
            pl.BlockSpec(memory_space=pltpu.SMEM),
        ),
    )(sums, weights, A)


def kernel(positions, net_to_pin, pin_to_macro, pin_offsets, rotation_onehot,
           net_weights):
    fbuf = jnp.concatenate([
        positions.astype(jnp.float32).reshape(-1),
        pin_offsets.astype(jnp.float32).reshape(-1),
        rotation_onehot.astype(jnp.float32).reshape(-1),
        lax.bitcast_convert_type(net_to_pin.astype(jnp.int32),
                                 jnp.float32).reshape(-1),
        lax.bitcast_convert_type(pin_to_macro.astype(jnp.int32),
                                 jnp.float32).reshape(-1),
    ])
    sums = _sc_stats()(fbuf)
    total, hpwl, penalty = _tc_tail(sums, net_weights, _GAUSS_A)
    return total, hpwl, penalty


# speedup vs baseline: 1.0250x; 1.0250x over previous
import functools

import numpy as np
import jax
import jax.numpy as jnp
from jax import lax
from jax.experimental import pallas as pl
from jax.experimental.pallas import tpu as pltpu
from jax.experimental.pallas import tpu_sc as plsc

_GAMMA = 10.0
_GRID = 64
_THRESH = 1.0
_SIGMA = 1.5
_CONG_W = 0.1
_STEEP = 2.0

_B, _V, _P, _N, _MP = 4, 512, 8192, 2048, 16
_NW = 32
_WPB = _NW // _B
_NPW = _N // _WPB
_NGRP = _NPW // 16
_CSH = float(_GRID // 2)


def _gauss_band_matrix():
    ksize = max(int(4 * _SIGMA) | 1, 3)
    half = ksize // 2
    x = np.arange(ksize, dtype=np.float64) - half
    g1 = np.exp(-(x ** 2) / (2 * _SIGMA ** 2))
    g1 = g1 / g1.sum()
    A = np.zeros((_GRID, _GRID), dtype=np.float32)
    for i in range(_GRID):
        for j in range(max(0, i - half), min(_GRID, i + half + 1)):
            A[i, j] = g1[j - i + half]
    return A


_GAUSS_A = jnp.asarray(_gauss_band_matrix())


def _sc_stats_body(ntp_hbm, p2m_hbm, off_hbm, pos_hbm, oh_hbm, out_hbm,
                   ntp_v, p2m_v, off_v, pos_v, oh_v, c_v, s_v, res_v):
    wid = lax.axis_index("s") * 2 + lax.axis_index("c")
    b = wid // _WPB
    n0 = (wid % _WPB) * _NPW

    pltpu.sync_copy(ntp_hbm.at[:, pl.ds(n0, _NPW)], ntp_v)
    pltpu.sync_copy(p2m_hbm, p2m_v)
    pltpu.sync_copy(off_hbm, off_v)
    pltpu.sync_copy(pos_hbm.at[pl.ds(b * _V * 2, _V * 2)], pos_v)
    pltpu.sync_copy(oh_hbm.at[pl.ds(b * _V * 4, _V * 4)], oh_v)

    lanes = lax.broadcasted_iota(jnp.int32, (16,), 0)
    neg = jnp.full((16,), -1e9, jnp.float32)
    pos = jnp.full((16,), 1e9, jnp.float32)

    @plsc.parallel_loop(0, _V // 16, unroll=2)
    def build_cs(j):
        i4 = (j * 16 + lanes) * 4
        o0 = plsc.load_gather(oh_v, [i4])
        o1 = plsc.load_gather(oh_v, [i4 + 1])
        o2 = plsc.load_gather(oh_v, [i4 + 2])
        o3 = plsc.load_gather(oh_v, [i4 + 3])
        sl = pl.ds(j * 16, 16)
        c_v[sl] = o0 - o2
        s_v[sl] = o1 - o3

    @plsc.parallel_loop(0, _NGRP, unroll=4)
    def group(t):
        nsl = pl.ds(t * 16, 16)
        zero = jnp.zeros((16,), jnp.float32)
        sgx, snx, sgy, sny = zero, zero, zero, zero
        bxmax, bymax = neg, neg
        bxmin, bymin = pos, pos
        for m in range(_MP):
            idx = ntp_v[m, nsl]
            valid = idx >= 0
            safe = jnp.maximum(idx, 0)
            mac = plsc.load_gather(p2m_v, [safe])
            mac2 = mac + mac
            px = plsc.load_gather(pos_v, [mac2])
            py = plsc.load_gather(pos_v, [mac2 + 1])
            c = plsc.load_gather(c_v, [mac])
            s = plsc.load_gather(s_v, [mac])
            safe2 = safe + safe
            ox = plsc.load_gather(off_v, [safe2])
            oy = plsc.load_gather(off_v, [safe2 + 1])
            x = px + c * ox - s * oy
            y = py + s * ox + c * oy
            xm = jnp.where(valid, x, neg)
            xn = jnp.where(valid, x, pos)
            ym = jnp.where(valid, y, neg)
            yn = jnp.where(valid, y, pos)
            sgx = sgx + jnp.exp(_GAMMA * xm)
            snx = snx + jnp.exp(-_GAMMA * xn)
            sgy = sgy + jnp.exp(_GAMMA * ym)
            sny = sny + jnp.exp(-_GAMMA * yn)
            bxmax = jnp.maximum(bxmax, xm)
            bxmin = jnp.minimum(bxmin, xn)
            bymax = jnp.maximum(bymax, ym)
            bymin = jnp.minimum(bymin, yn)
        res_v[0, nsl] = sgx
        res_v[1, nsl] = snx
        res_v[2, nsl] = sgy
        res_v[3, nsl] = sny
        res_v[4, nsl] = bxmax
        res_v[5, nsl] = bxmin
        res_v[6, nsl] = bymax
        res_v[7, nsl] = bymin

    for k in range(8):
        pltpu.sync_copy(res_v.at[k], out_hbm.at[b * 8 + k, pl.ds(n0, _NPW)])


@functools.lru_cache(maxsize=1)
def _sc_stats():
    return pl.kernel(
        _sc_stats_body,
        mesh=plsc.VectorSubcoreMesh(core_axis_name="c", subcore_axis_name="s"),
        compiler_params=pltpu.CompilerParams(needs_layout_passes=False),
        out_type=jax.ShapeDtypeStruct((8 * _B, _N), jnp.float32),
        scratch_types=[
            pltpu.VMEM((_MP, _NPW), jnp.int32),
            pltpu.VMEM((_P,), jnp.int32),
            pltpu.VMEM((_P * 2,), jnp.float32),
            pltpu.VMEM((_V * 2,), jnp.float32),
            pltpu.VMEM((_V * 4,), jnp.float32),
            pltpu.VMEM((_V,), jnp.float32),
            pltpu.VMEM((_V,), jnp.float32),
            pltpu.VMEM((8, _NPW), jnp.float32),
        ],
    )


def _tc_tail_body(sums_ref, w_ref, A_ref, tot_ref, hpwl_ref, pen_ref):
    A = A_ref[...]
    w = w_ref[...]
    irow = lax.broadcasted_iota(jnp.int32, (_GRID, 1), 0).astype(jnp.float32)
    u = jnp.exp(_STEEP * (_CSH - irow))
    v = jnp.exp(_STEEP * (irow - _CSH))

    for b in range(_B):
        S = sums_ref[b * 8:(b + 1) * 8, :]
        logs = jnp.log(S[0:4, :])
        wl = jnp.sum(logs, axis=0, keepdims=True) * (1.0 / _GAMMA)
        hp = jnp.sum(wl * w)

        scale = 0.5 * (_GRID - 1)
        gxmax = (S[4:5, :] + 1.0) * scale
        gxmin = (S[5:6, :] + 1.0) * scale
        gymax = (S[6:7, :] + 1.0) * scale
        gymin = (S[7:8, :] + 1.0) * scale

        eax = jnp.exp(_STEEP * (gxmin - _CSH))
        ebx = jnp.exp(-_STEEP * (gxmax - _CSH))
        eay = jnp.exp(_STEEP * (gymin - _CSH))
        eby = jnp.exp(-_STEEP * (gymax - _CSH))
        wx = 1.0 / ((1.0 + eax * ebx) + (eax * u + ebx * v))
        wy = 1.0 / ((1.0 + eay * eby) + (eay * u + eby * v))
        area = jnp.clip((gxmax - gxmin + 1.0) * (gymax - gymin + 1.0),
                        1.0, None)
        wxa = wx * (1.0 / area)

        rudy = lax.dot_general(
            wy, wxa, (((1,), (1,)), ((), ())),
            preferred_element_type=jnp.float32,
            precision=lax.Precision.HIGHEST)
        sm = jnp.dot(A, rudy, preferred_element_type=jnp.float32,
                     precision=lax.Precision.HIGHEST)
        sm = jnp.dot(sm, A, preferred_element_type=jnp.float32,
                     precision=lax.Precision.HIGHEST)
        o = jnp.maximum(sm - _THRESH, 0.0)
        pen = jnp.sum(o * o)

        hpwl_ref[b] = hp
        pen_ref[b] = pen
        tot_ref[b] = hp + _CONG_W * pen


def _tc_tail(sums, weights_row, A):
    return pl.pallas_call(
        _tc_tail_body,
        out_shape=(
            jax.ShapeDtypeStruct((_B,), jnp.float32),
            jax.ShapeDtypeStruct((_B,), jnp.float32),
            jax.ShapeDtypeStruct((_B,), jnp.float32),
        ),
        in_specs=[
            pl.BlockSpec(memory_space=pltpu.VMEM),
            pl.BlockSpec(memory_space=pltpu.VMEM),
            pl.BlockSpec(memory_space=pltpu.VMEM),
        ],
        out_specs=(
            pl.BlockSpec(memory_space=pltpu.SMEM),
            pl.BlockSpec(memory_space=pltpu.SMEM),
            pl.BlockSpec(memory_space=pltpu.SMEM),
        ),
    )(sums, weights_row, A)


def kernel(positions, net_to_pin, pin_to_macro, pin_offsets, rotation_onehot,
           net_weights):
    ntp = net_to_pin.astype(jnp.int32).T
    p2m = pin_to_macro.astype(jnp.int32)
    off = pin_offsets.astype(jnp.float32).reshape(-1)
    posf = positions.astype(jnp.float32).reshape(-1)
    ohf = rotation_onehot.astype(jnp.float32).reshape(-1)

    sums = _sc_stats()(ntp, p2m, off, posf, ohf)

    total, hpwl, penalty = _tc_tail(sums, net_weights.reshape(1, _N),
                                    _GAUSS_A)
    return total, hpwl, penalty
